# Initial kernel scaffold; baseline (speedup 1.0000x reference)
import jax
import jax.numpy as jnp
from jax.experimental import pallas as pl


def kernel(x, W):
    raise NotImplementedError("write your pallas kernel here")


# speedup vs baseline: 1.0845x; 1.0845x over previous
import functools

import jax
import jax.numpy as jnp
from jax import lax
from jax.experimental import pallas as pl
from jax.experimental.pallas import tpu as pltpu
from jax.experimental.pallas import tpu_sc as plsc

N = 16384
K = 1024
D = 64
BN = 2048
NB = N // BN

NW = 32
BPW = N // NW
CHUNK = 128
NCH = BPW // CHUNK


def _index_body(x_ref, wt_ref, idx_ref):
    x = x_ref[...]
    wt = wt_ref[...]
    y = x * x
    p = y[:, 0:8]
    for k in range(1, 8):
        p = p + y[:, 8 * k:8 * k + 8]
    t1 = p[:, 0:4] + p[:, 4:8]
    t2 = t1[:, 0:2] + t1[:, 2:4]
    x2 = t2[:, 0:1] + t2[:, 1:2]
    w2 = jnp.sum(wt * wt, axis=0, keepdims=True)
    xw = lax.dot_general(x, wt, (((1,), (0,)), ((), ())),
                         preferred_element_type=jnp.float32)
    dist = jnp.sqrt(jnp.maximum(x2 - 2.0 * xw + w2, 0.0))
    m = jnp.min(dist, axis=1, keepdims=True)
    iota = lax.broadcasted_iota(jnp.int32, (BN, K), 1)
    cand = jnp.where(dist == m, iota, K)
    idx_ref[...] = jnp.min(cand, axis=1, keepdims=True)


def _compute_indices(x, wt):
    out = pl.pallas_call(
        _index_body,
        grid=(NB,),
        in_specs=[
            pl.BlockSpec((BN, D), lambda i: (i, 0)),
            pl.BlockSpec((D, K), lambda i: (0, 0)),
        ],
        out_specs=pl.BlockSpec((BN, 1), lambda i: (i, 0)),
        out_shape=jax.ShapeDtypeStruct((N, 1), jnp.int32),
    )(x, wt)
    return out.reshape(N)


def _gather_rows(W, idx3):
    mesh = plsc.VectorSubcoreMesh(core_axis_name="c", subcore_axis_name="s")

    @functools.partial(
        pl.kernel,
        mesh=mesh,
        out_type=jax.ShapeDtypeStruct((N, D), jnp.float32),
        scratch_types=[
            pltpu.VMEM((NCH, CHUNK), jnp.int32),
            pltpu.VMEM((BPW, D), jnp.float32),
            pltpu.SemaphoreType.DMA,
        ],
        compiler_params=pltpu.CompilerParams(use_tc_tiling_on_sc=False),
    )
    def gk(table_hbm, idx_hbm, out_hbm, idx_v, rows_v, sem):
        wid = lax.axis_index("s") * 2 + lax.axis_index("c")
        base = wid * BPW
        pltpu.sync_copy(idx_hbm.at[wid], idx_v)
        for j in range(NCH):
            pltpu.async_copy(
                table_hbm.at[idx_v.at[j]],
                rows_v.at[pl.ds(j * CHUNK, CHUNK)],
                sem,
            ).wait()
        pltpu.sync_copy(rows_v, out_hbm.at[pl.ds(base, BPW)])

    return gk(W, idx3)


def kernel(x, W):
    wt = W.T
    indices = _compute_indices(x, wt)
    z = _gather_rows(W, indices.reshape(NW, NCH, CHUNK))
    return (z, z, x, indices)


# speedup vs baseline: 1.5567x; 1.4354x over previous
import functools

import jax
import jax.numpy as jnp
from jax import lax
from jax.experimental import pallas as pl
from jax.experimental.pallas import tpu as pltpu
from jax.experimental.pallas import tpu_sc as plsc

N = 16384
K = 1024
D = 64
BN = 2048
NB = N // BN

NW = 32
BPW = N // NW
CHUNK = 128
NCH = BPW // CHUNK


def _index_body(w_ref, xt_ref, wt_ref, idx_ref):
    Wm = w_ref[...]
    xtv = xt_ref[...]
    wtv = wt_ref[...]

    s = lax.dot_general(Wm * -2.0, xtv, (((1,), (0,)), ((), ())),
                        preferred_element_type=jnp.float32)

    y = xtv * xtv
    p = y[0:8, :]
    for k in range(1, 8):
        p = p + y[8 * k:8 * k + 8, :]
    t1 = p[0:4, :] + p[4:8, :]
    t2 = t1[0:2, :] + t1[2:4, :]
    x2 = t2[0:1, :] + t2[1:2, :]

    w2col = jnp.sum(wtv * wtv, axis=0, keepdims=True).T

    dist = jnp.sqrt(jnp.maximum((x2 + s) + w2col, 0.0))
    m = jnp.min(dist, axis=0, keepdims=True)
    iota = lax.broadcasted_iota(jnp.int32, (K, 1), 0).astype(jnp.float32)
    cand = jnp.where(dist == m, iota, float(K))
    idxf = jnp.min(cand, axis=0, keepdims=True)
    idx_ref[...] = idxf.astype(jnp.int32).reshape(1, 1, BN)


def _compute_indices(W, xt, wt):
    out = pl.pallas_call(
        _index_body,
        grid=(NB,),
        in_specs=[
            pl.BlockSpec((K, D), lambda i: (0, 0)),
            pl.BlockSpec((D, BN), lambda i: (0, i)),
            pl.BlockSpec((D, K), lambda i: (0, 0)),
        ],
        out_specs=pl.BlockSpec((1, 1, BN), lambda i: (i, 0, 0)),
        out_shape=jax.ShapeDtypeStruct((NB, 1, BN), jnp.int32),
    )(W, xt, wt)
    return out.reshape(N)


def _gather_rows(W, idx3):
    mesh = plsc.VectorSubcoreMesh(core_axis_name="c", subcore_axis_name="s")

    @functools.partial(
        pl.kernel,
        mesh=mesh,
        out_type=jax.ShapeDtypeStruct((N, D), jnp.float32),
        scratch_types=[
            pltpu.VMEM((NCH, CHUNK), jnp.int32),
            pltpu.VMEM((BPW, D), jnp.float32),
            pltpu.SemaphoreType.DMA,
        ],
        compiler_params=pltpu.CompilerParams(use_tc_tiling_on_sc=False),
    )
    def gk(table_hbm, idx_hbm, out_hbm, idx_v, rows_v, sem):
        wid = lax.axis_index("s") * 2 + lax.axis_index("c")
        base = wid * BPW
        pltpu.sync_copy(idx_hbm.at[wid], idx_v)
        for j in range(NCH):
            pltpu.async_copy(
                table_hbm.at[idx_v.at[j]],
                rows_v.at[pl.ds(j * CHUNK, CHUNK)],
                sem,
            ).wait()
        pltpu.sync_copy(rows_v, out_hbm.at[pl.ds(base, BPW)])

    return gk(W, idx3)


def kernel(x, W):
    indices = _compute_indices(W, x.T, W.T)
    z = _gather_rows(W, indices.reshape(NW, NCH, CHUNK))
    return (z, z, x, indices)


# speedup vs baseline: 1.5880x; 1.0201x over previous
import functools

import jax
import jax.numpy as jnp
from jax import lax
from jax.experimental import pallas as pl
from jax.experimental.pallas import tpu as pltpu
from jax.experimental.pallas import tpu_sc as plsc

N = 16384
K = 1024
D = 64
BN = 4096
NB = N // BN

NW = 32
BPW = N // NW
CHUNK = 128
NCH = BPW // CHUNK


def _index_body(w_ref, xt_ref, wt_ref, idx_ref):
    Wm = w_ref[...]
    xtv = xt_ref[...]
    wtv = wt_ref[...]

    s = lax.dot_general(Wm * -2.0, xtv, (((1,), (0,)), ((), ())),
                        preferred_element_type=jnp.float32)

    y = xtv * xtv
    p = y[0:8, :]
    for k in range(1, 8):
        p = p + y[8 * k:8 * k + 8, :]
    t1 = p[0:4, :] + p[4:8, :]
    t2 = t1[0:2, :] + t1[2:4, :]
    x2 = t2[0:1, :] + t2[1:2, :]

    w2col = jnp.sum(wtv * wtv, axis=0, keepdims=True).T

    dist = jnp.sqrt(jnp.maximum((x2 + s) + w2col, 0.0))
    m = jnp.min(dist, axis=0, keepdims=True)
    iota = lax.broadcasted_iota(jnp.int32, (K, 1), 0).astype(jnp.float32)
    cand = jnp.where(dist == m, iota, float(K))
    idxf = jnp.min(cand, axis=0, keepdims=True)
    idx_ref[...] = idxf.astype(jnp.int32).reshape(1, 1, BN)


def _compute_indices(W, xt, wt):
    out = pl.pallas_call(
        _index_body,
        grid=(NB,),
        in_specs=[
            pl.BlockSpec((K, D), lambda i: (0, 0)),
            pl.BlockSpec((D, BN), lambda i: (0, i)),
            pl.BlockSpec((D, K), lambda i: (0, 0)),
        ],
        out_specs=pl.BlockSpec((1, 1, BN), lambda i: (i, 0, 0)),
        out_shape=jax.ShapeDtypeStruct((NB, 1, BN), jnp.int32),
    )(W, xt, wt)
    return out.reshape(N)


def _gather_rows(W, idx3):
    mesh = plsc.VectorSubcoreMesh(core_axis_name="c", subcore_axis_name="s")

    @functools.partial(
        pl.kernel,
        mesh=mesh,
        out_type=jax.ShapeDtypeStruct((N, D), jnp.float32),
        scratch_types=[
            pltpu.VMEM((NCH, CHUNK), jnp.int32),
            pltpu.VMEM((BPW, D), jnp.float32),
            pltpu.SemaphoreType.DMA,
        ],
        compiler_params=pltpu.CompilerParams(use_tc_tiling_on_sc=False),
    )
    def gk(table_hbm, idx_hbm, out_hbm, idx_v, rows_v, sem):
        wid = lax.axis_index("s") * 2 + lax.axis_index("c")
        base = wid * BPW
        pltpu.sync_copy(idx_hbm.at[wid], idx_v)
        for j in range(NCH):
            pltpu.async_copy(
                table_hbm.at[idx_v.at[j]],
                rows_v.at[pl.ds(j * CHUNK, CHUNK)],
                sem,
            ).wait()
        pltpu.sync_copy(rows_v, out_hbm.at[pl.ds(base, BPW)])

    return gk(W, idx3)


def kernel(x, W):
    indices = _compute_indices(W, x.T, W.T)
    z = _gather_rows(W, indices.reshape(NW, NCH, CHUNK))
    return (z, z, x, indices)


# speedup vs baseline: 1.6092x; 1.0133x over previous
import functools

import jax
import jax.numpy as jnp
from jax import lax
from jax.experimental import pallas as pl
from jax.experimental.pallas import tpu as pltpu
from jax.experimental.pallas import tpu_sc as plsc

N = 16384
K = 1024
D = 64
BN = 4096
NB = N // BN

NW = 32
BPW = N // NW
CHUNK = 128
NCH = BPW // CHUNK


def _index_body(w_ref, xt_ref, wt_ref, idx_ref):
    Wm = w_ref[...]
    xtv = xt_ref[...]
    wtv = wt_ref[...]

    s = lax.dot_general(Wm * -2.0, xtv, (((1,), (0,)), ((), ())),
                        preferred_element_type=jnp.float32)

    y = xtv * xtv
    p = y[0:8, :]
    for k in range(1, 8):
        p = p + y[8 * k:8 * k + 8, :]
    t1 = p[0:4, :] + p[4:8, :]
    t2 = t1[0:2, :] + t1[2:4, :]
    x2 = t2[0:1, :] + t2[1:2, :]

    w2col = jnp.sum(wtv * wtv, axis=0, keepdims=True).T

    dist = jnp.sqrt(jnp.maximum((x2 + s) + w2col, 0.0))
    m = jnp.min(dist, axis=0, keepdims=True)
    iota = lax.broadcasted_iota(jnp.int32, (K, 1), 0).astype(jnp.float32)
    cand = jnp.where(dist == m, iota, float(K))
    idxf = jnp.min(cand, axis=0, keepdims=True)
    idx_ref[...] = idxf.astype(jnp.int32).reshape(1, 1, BN)


def _compute_indices(W, xt, wt):
    out = pl.pallas_call(
        _index_body,
        grid=(NB,),
        in_specs=[
            pl.BlockSpec((K, D), lambda i: (0, 0)),
            pl.BlockSpec((D, BN), lambda i: (0, i)),
            pl.BlockSpec((D, K), lambda i: (0, 0)),
        ],
        out_specs=pl.BlockSpec((1, 1, BN), lambda i: (i, 0, 0)),
        out_shape=jax.ShapeDtypeStruct((NB, 1, BN), jnp.int32),
    )(W, xt, wt)
    return out.reshape(N)


def _gather_rows(W, idx3):
    mesh = plsc.VectorSubcoreMesh(core_axis_name="c", subcore_axis_name="s")

    @functools.partial(
        pl.kernel,
        mesh=mesh,
        out_type=jax.ShapeDtypeStruct((N, D), jnp.float32),
        scratch_types=[
            pltpu.VMEM((NCH, CHUNK), jnp.int32),
            pltpu.VMEM((BPW, D), jnp.float32),
            pltpu.SemaphoreType.DMA,
        ],
        compiler_params=pltpu.CompilerParams(use_tc_tiling_on_sc=False),
    )
    def gk(table_hbm, idx_hbm, out_hbm, idx_v, rows_v, sem):
        wid = lax.axis_index("s") * 2 + lax.axis_index("c")
        base = wid * BPW
        pltpu.sync_copy(idx_hbm.at[wid], idx_v)
        cps = [
            pltpu.async_copy(
                table_hbm.at[idx_v.at[j]],
                rows_v.at[pl.ds(j * CHUNK, CHUNK)],
                sem,
            )
            for j in range(NCH)
        ]
        for c in cps:
            c.wait()
        pltpu.sync_copy(rows_v, out_hbm.at[pl.ds(base, BPW)])

    return gk(W, idx3)


def kernel(x, W):
    indices = _compute_indices(W, x.T, W.T)
    z = _gather_rows(W, indices.reshape(NW, NCH, CHUNK))
    return (z, z, x, indices)
